# Initial kernel scaffold; baseline (speedup 1.0000x reference)
import functools

import jax
import jax.numpy as jnp
from jax import lax
from jax.experimental import pallas as pl
from jax.experimental.pallas import tpu as pltpu
from jax.experimental.pallas import tpu_sc as plsc

B_, T_IN, N_, C_IN = 32, 40, 50, 1
HID, OUT, K_, E_ = 32, 64, 3, 800
NP = 64
G_ = B_ * (T_IN - 2)
NW = 32
GPW = G_ // NW
GC = 32
NCHUNK = G_ // GC
F32 = jnp.float32
_PREC = lax.Precision.HIGHEST


def _build_adj(ei_flat):
    mesh = plsc.VectorSubcoreMesh(core_axis_name="c", subcore_axis_name="s")

    @functools.partial(
        pl.kernel,
        mesh=mesh,
        out_type=(
            jax.ShapeDtypeStruct((G_, NP * NP), F32),
            jax.ShapeDtypeStruct((G_, NP), F32),
        ),
        scratch_types=[
            pltpu.VMEM((E_,), jnp.int32),
            pltpu.VMEM((E_,), jnp.int32),
            pltpu.VMEM((NP * NP,), F32),
            pltpu.VMEM((NP,), F32),
            pltpu.VMEM((NP * NP,), F32),
        ],
    )
    def adj_kernel(ei_hbm, b_hbm, deg_hbm, rows_v, cols_v, bloc_v, deg_v, zer_v):
        wid = lax.axis_index("s") * 2 + lax.axis_index("c")
        zero16 = jnp.zeros((16,), F32)

        @pl.loop(0, NP * NP, step=16)
        def _(i):
            zer_v[pl.ds(i, 16)] = zero16

        @pl.loop(0, GPW)
        def _(j):
            g = wid * GPW + j
            pltpu.sync_copy(ei_hbm.at[g, 0], rows_v)
            pltpu.sync_copy(ei_hbm.at[g, 1], cols_v)
            pltpu.sync_copy(zer_v, bloc_v)

            @pl.loop(0, NP, step=16)
            def _(i):
                deg_v[pl.ds(i, 16)] = zero16

            @pl.loop(0, E_, step=16)
            def _(e):
                r = rows_v[pl.ds(e, 16)]
                c = cols_v[pl.ds(e, 16)]
                ew = jnp.where(r != c, 1.0, 0.0).astype(F32)
                plsc.addupdate_scatter(deg_v, [r], ew)
                plsc.addupdate_scatter(bloc_v, [c * NP + r], ew)

            pltpu.sync_copy(bloc_v, b_hbm.at[g])
            pltpu.sync_copy(deg_v, deg_hbm.at[g])

    return adj_kernel(ei_flat)


def _k1_body(x_ref, w_ref, b_ref, out_ref):
    x2 = x_ref[0]
    acc = b_ref[...].reshape(96, 1, 1) * jnp.ones((96, T_IN - 2, NP), F32)
    for dt in range(3):
        xs = x2[dt:dt + T_IN - 2][None, :, :]
        acc = acc + w_ref[:, dt:dt + 1].reshape(96, 1, 1) * xs
    p = acc[0:HID]
    q = acc[HID:2 * HID]
    r = acc[2 * HID:3 * HID]
    h = jax.nn.relu(p * jax.nn.sigmoid(q) + r)
    out_ref[...] = h.reshape(HID, 1, T_IN - 2, NP)


def _tconv1(xp, w3, b3):
    return pl.pallas_call(
        _k1_body,
        grid=(B_,),
        in_specs=[
            pl.BlockSpec((1, T_IN, NP), lambda b: (b, 0, 0)),
            pl.BlockSpec((96, 3), lambda b: (0, 0)),
            pl.BlockSpec((96, 1), lambda b: (0, 0)),
        ],
        out_specs=pl.BlockSpec((HID, 1, T_IN - 2, NP), lambda b: (0, b, 0, 0)),
        out_shape=jax.ShapeDtypeStruct((HID, B_, T_IN - 2, NP), F32),
        compiler_params=pltpu.CompilerParams(dimension_semantics=("parallel",)),
    )(xp, w3, b3)


def _k2_body(x_ref, b_ref, deg_ref, w_ref, cb_ref, out_ref):
    x_t = x_ref[:, 0, :]
    u = jnp.dot(w_ref[...], x_t, preferred_element_type=F32,
                precision=_PREC)
    bmat = b_ref[0].reshape(GC, NP, NP)
    deg = deg_ref[0]
    dis = jnp.where(deg > 0, lax.rsqrt(jnp.where(deg > 0, deg, 1.0)), 0.0)
    dis_g = dis[:, None, :]

    def to_g(z):
        return jnp.transpose(z.reshape(HID, GC, NP), (1, 0, 2))

    u0, u1, u2 = to_g(u[0:HID]), to_g(u[HID:2 * HID]), to_g(u[2 * HID:])
    dn = (((2,), (1,)), ((0,), (0,)))
    p2 = lax.dot_general(u2 * dis_g, bmat, dn, preferred_element_type=F32,
                         precision=_PREC)
    v = u1 - 2.0 * dis_g * p2
    p1 = lax.dot_general(v * dis_g, bmat, dn, preferred_element_type=F32,
                         precision=_PREC)
    outg = jax.nn.relu(u0 - dis_g * p1 + cb_ref[...].reshape(1, HID, 1))
    out_ref[:, 0, :] = jnp.transpose(outg, (1, 0, 2)).reshape(HID, GC * NP)


def _cheb(t0t, bcounts, deg, wcat, cb):
    t0v = t0t.reshape(HID, NCHUNK, GC * NP)
    bv = bcounts.reshape(NCHUNK, GC, NP * NP)
    degv = deg.reshape(NCHUNK, GC, NP)
    return pl.pallas_call(
        _k2_body,
        grid=(NCHUNK,),
        in_specs=[
            pl.BlockSpec((HID, 1, GC * NP), lambda i: (0, i, 0)),
            pl.BlockSpec((1, GC, NP * NP), lambda i: (i, 0, 0)),
            pl.BlockSpec((1, GC, NP), lambda i: (i, 0, 0)),
            pl.BlockSpec((3 * HID, HID), lambda i: (0, 0)),
            pl.BlockSpec((HID, 1), lambda i: (0, 0)),
        ],
        out_specs=pl.BlockSpec((HID, 1, GC * NP), lambda i: (0, i, 0)),
        out_shape=jax.ShapeDtypeStruct((HID, NCHUNK, GC * NP), F32),
        compiler_params=pltpu.CompilerParams(dimension_semantics=("parallel",)),
    )(t0v, bv, degv, wcat, cb)


def _k3_body(x_ref, w_ref, b_ref, out_ref):
    x = x_ref[:, 0]
    x2 = jnp.concatenate([x[:, 0:36], x[:, 1:37], x[:, 2:38]], axis=0)
    x2 = x2.reshape(3 * HID, 36 * NP)
    y = jnp.dot(w_ref[...], x2, preferred_element_type=F32,
                precision=_PREC) + b_ref[...].reshape(192, 1)
    p = y[0:OUT]
    q = y[OUT:2 * OUT]
    r = y[2 * OUT:]
    t2 = jax.nn.relu(p * jax.nn.sigmoid(q) + r)
    out_ref[:, 0] = t2.reshape(OUT, 36, NP)


def _tconv2(t1t, w2all, b2):
    t1v = t1t.reshape(HID, B_, T_IN - 2, NP)
    return pl.pallas_call(
        _k3_body,
        grid=(B_,),
        in_specs=[
            pl.BlockSpec((HID, 1, T_IN - 2, NP), lambda b: (0, b, 0, 0)),
            pl.BlockSpec((192, 96), lambda b: (0, 0)),
            pl.BlockSpec((192, 1), lambda b: (0, 0)),
        ],
        out_specs=pl.BlockSpec((OUT, 1, 36, NP), lambda b: (0, b, 0, 0)),
        out_shape=jax.ShapeDtypeStruct((OUT, B_, 36, NP), F32),
        compiler_params=pltpu.CompilerParams(dimension_semantics=("parallel",)),
    )(t1v, w2all, b2)


def _k4a_body(x_ref, out_ref):
    b = pl.program_id(0)

    @pl.when(b == 0)
    def _():
        out_ref[...] = jnp.zeros_like(out_ref)

    x = x_ref[:, 0]
    s = jnp.sum(x, axis=(0, 1))
    ss = jnp.sum(x * x, axis=(0, 1))
    out_ref[0:1, :] += s[None, :]
    out_ref[1:2, :] += ss[None, :]


def _bn_stats(t2t):
    return pl.pallas_call(
        _k4a_body,
        grid=(B_,),
        in_specs=[pl.BlockSpec((OUT, 1, 36, NP), lambda b: (0, b, 0, 0))],
        out_specs=pl.BlockSpec((8, NP), lambda b: (0, 0)),
        out_shape=jax.ShapeDtypeStruct((8, NP), F32),
        compiler_params=pltpu.CompilerParams(dimension_semantics=("arbitrary",)),
    )(t2t)


def _k4b_body(x_ref, st_ref, bn_ref, out_ref):
    cnt = float(B_ * 36 * OUT)
    mean = st_ref[0:1, :] / cnt
    var = st_ref[1:2, :] / cnt - mean * mean
    gamma = bn_ref[0:1, :]
    beta = bn_ref[1:2, :]
    scale = gamma * lax.rsqrt(var + 1e-5)
    shift = beta - mean * scale
    x = x_ref[:, 0]
    xn = x * scale.reshape(1, 1, NP) + shift.reshape(1, 1, NP)
    s_all = jnp.sum(xn, axis=1)
    a0 = (s_all - xn[:, 34] - xn[:, 35]) * (1.0 / 34.0)
    a1 = (s_all - xn[:, 0] - xn[:, 35]) * (1.0 / 34.0)
    a2 = (s_all - xn[:, 0] - xn[:, 1]) * (1.0 / 34.0)
    a3 = jnp.stack([a0.T, a1.T, a2.T], axis=0)
    out_ref[0, 0] = a3[:, 0:N_, :].reshape(3 * N_ * OUT)


def _bn_pool(t2t, stats, bnp):
    return pl.pallas_call(
        _k4b_body,
        grid=(B_,),
        in_specs=[
            pl.BlockSpec((OUT, 1, 36, NP), lambda b: (0, b, 0, 0)),
            pl.BlockSpec((8, NP), lambda b: (0, 0)),
            pl.BlockSpec((8, NP), lambda b: (0, 0)),
        ],
        out_specs=pl.BlockSpec((1, 1, 3 * N_ * OUT), lambda b: (b, 0, 0)),
        out_shape=jax.ShapeDtypeStruct((B_, 1, 3 * N_ * OUT), F32),
        compiler_params=pltpu.CompilerParams(dimension_semantics=("parallel",)),
    )(t2t, stats, bnp)


def _k4c_body(a_ref, wc_ref, cb_ref, fw_ref, fb_ref, out_ref):
    pooled = jnp.dot(a_ref[...], wc_ref[...], preferred_element_type=F32,
                     precision=_PREC) + cb_ref[...]
    out_ref[...] = jnp.dot(pooled, fw_ref[...], preferred_element_type=F32,
                           precision=_PREC) + fb_ref[...]


def _final(a3v, wc, c3b, f1_w, f1_b):
    return pl.pallas_call(
        _k4c_body,
        out_shape=jax.ShapeDtypeStruct((B_, N_ * OUT), F32),
    )(a3v, wc, c3b, f1_w, f1_b)


def kernel(X, edge_index, tc1_w1, tc1_b1, tc1_w2, tc1_b2, tc1_w3, tc1_b3,
           cheb_W, cheb_b, tc2_w1, tc2_b1, tc2_w2, tc2_b2, tc2_w3, tc2_b3,
           bn_gamma, bn_beta, conv3_w, conv3_b, f1_w, f1_b):
    ei_flat = edge_index.reshape(G_, 2, E_)
    bcounts, deg = _build_adj(ei_flat)

    xp = jnp.pad(X[..., 0], ((0, 0), (0, 0), (0, NP - N_)))
    w3 = jnp.concatenate([tc1_w1[:, 0, 0, :], tc1_w2[:, 0, 0, :],
                          tc1_w3[:, 0, 0, :]], axis=0)
    b3 = jnp.concatenate([tc1_b1, tc1_b2, tc1_b3])[:, None]
    t0t = _tconv1(xp, w3, b3)

    wcat = jnp.concatenate([(cheb_W[0] - cheb_W[2]).T, cheb_W[1].T,
                            2.0 * cheb_W[2].T], axis=0)
    t1t = _cheb(t0t.reshape(HID, G_ * NP), bcounts, deg, wcat,
                cheb_b[:, None])

    def _w2(w):
        return jnp.transpose(w[:, :, 0, :], (2, 1, 0)).reshape(3 * HID, OUT).T

    w2all = jnp.concatenate([_w2(tc2_w1), _w2(tc2_w2), _w2(tc2_w3)], axis=0)
    b2 = jnp.concatenate([tc2_b1, tc2_b2, tc2_b3])[:, None]
    t2t = _tconv2(t1t.reshape(HID, G_ * NP), w2all, b2)

    stats = _bn_stats(t2t)
    bnp = jnp.zeros((8, NP), F32)
    bnp = bnp.at[0, :N_].set(bn_gamma).at[1, :N_].set(bn_beta)
    a3v = _bn_pool(t2t, stats, bnp)

    wc = conv3_w.reshape(128, 3 * N_ * OUT).T
    return _final(a3v.reshape(B_, 3 * N_ * OUT), wc, conv3_b[None, :],
                  f1_w, f1_b[None, :])


# speedup vs baseline: 143.3902x; 143.3902x over previous
import dataclasses
import functools

import jax
import jax.numpy as jnp
from jax import lax
from jax.experimental import pallas as pl
from jax.experimental.pallas import tpu as pltpu
from jax.experimental.pallas import tpu_sc as plsc

B_, T_IN, N_, C_IN = 32, 40, 50, 1
HID, OUT, K_, E_ = 32, 64, 3, 800
NP = 64
G_ = B_ * (T_IN - 2)
NW = 32
GPW = G_ // NW
GC = 32
NCHUNK = G_ // GC
F32 = jnp.float32
_PREC = lax.Precision.HIGHEST


def _build_adj(ei_flat):
    mesh = plsc.VectorSubcoreMesh(core_axis_name="c", subcore_axis_name="s")
    cp = pltpu.CompilerParams()
    if "needs_layout_passes" in pltpu.CompilerParams.__dataclass_fields__:
        cp = dataclasses.replace(cp, needs_layout_passes=False)

    @functools.partial(
        pl.kernel,
        mesh=mesh,
        compiler_params=cp,
        out_type=(
            jax.ShapeDtypeStruct((G_, NP * NP), F32),
            jax.ShapeDtypeStruct((G_, NP), F32),
        ),
        scratch_types=[
            pltpu.VMEM((E_,), jnp.int32),
            pltpu.VMEM((E_,), jnp.int32),
            pltpu.VMEM((NP * NP,), F32),
            pltpu.VMEM((NP,), F32),
        ],
    )
    def adj_kernel(ei_hbm, b_hbm, deg_hbm, rows_v, cols_v, bloc_v, deg_v):
        wid = lax.axis_index("s") * 2 + lax.axis_index("c")
        zero16 = jnp.zeros((16,), F32)

        @pl.loop(0, GPW)
        def _(j):
            g = wid * GPW + j
            pltpu.sync_copy(ei_hbm.at[g, 0], rows_v)
            pltpu.sync_copy(ei_hbm.at[g, 1], cols_v)

            @pl.loop(0, NP * NP, step=16)
            def _(i):
                bloc_v[pl.ds(i, 16)] = zero16

            @pl.loop(0, NP, step=16)
            def _(i):
                deg_v[pl.ds(i, 16)] = zero16

            @pl.loop(0, E_, step=16)
            def _(e):
                r = rows_v[pl.ds(e, 16)]
                c = cols_v[pl.ds(e, 16)]
                ew = jnp.where(r != c, 1.0, 0.0).astype(F32)
                plsc.addupdate_scatter(deg_v, [r], ew)
                plsc.addupdate_scatter(bloc_v, [c * NP + r], ew)

            pltpu.sync_copy(bloc_v, b_hbm.at[g])
            pltpu.sync_copy(deg_v, deg_hbm.at[g])

    return adj_kernel(ei_flat)


def _k1_body(x_ref, w_ref, b_ref, out_ref):
    x2 = x_ref[0]
    acc = b_ref[...].reshape(96, 1, 1) * jnp.ones((96, T_IN - 2, NP), F32)
    for dt in range(3):
        xs = x2[dt:dt + T_IN - 2][None, :, :]
        acc = acc + w_ref[:, dt:dt + 1].reshape(96, 1, 1) * xs
    p = acc[0:HID]
    q = acc[HID:2 * HID]
    r = acc[2 * HID:3 * HID]
    h = jax.nn.relu(p * jax.nn.sigmoid(q) + r)
    out_ref[...] = h.reshape(HID, 1, T_IN - 2, NP)


def _tconv1(xp, w3, b3):
    return pl.pallas_call(
        _k1_body,
        grid=(B_,),
        in_specs=[
            pl.BlockSpec((1, T_IN, NP), lambda b: (b, 0, 0)),
            pl.BlockSpec((96, 3), lambda b: (0, 0)),
            pl.BlockSpec((96, 1), lambda b: (0, 0)),
        ],
        out_specs=pl.BlockSpec((HID, 1, T_IN - 2, NP), lambda b: (0, b, 0, 0)),
        out_shape=jax.ShapeDtypeStruct((HID, B_, T_IN - 2, NP), F32),
        compiler_params=pltpu.CompilerParams(dimension_semantics=("parallel",)),
    )(xp, w3, b3)


def _k2_body(x_ref, b_ref, deg_ref, w_ref, cb_ref, out_ref):
    x_t = x_ref[:, 0, 0, :]
    u = jnp.dot(w_ref[...], x_t, preferred_element_type=F32,
                precision=_PREC)
    bmat = b_ref[0].reshape(GC, NP, NP)
    deg = deg_ref[0]
    dis = jnp.where(deg > 0, lax.rsqrt(jnp.where(deg > 0, deg, 1.0)), 0.0)
    dis_g = dis[:, None, :]

    def to_g(z):
        return jnp.transpose(z.reshape(HID, GC, NP), (1, 0, 2))

    u0, u1, u2 = to_g(u[0:HID]), to_g(u[HID:2 * HID]), to_g(u[2 * HID:])
    dn = (((2,), (1,)), ((0,), (0,)))
    p2 = lax.dot_general(u2 * dis_g, bmat, dn, preferred_element_type=F32,
                         precision=_PREC)
    v = u1 - dis_g * p2
    p1 = lax.dot_general(v * dis_g, bmat, dn, preferred_element_type=F32,
                         precision=_PREC)
    outg = jax.nn.relu(u0 - dis_g * p1 + cb_ref[...].reshape(1, HID, 1))
    out_ref[:, 0, 0, :] = jnp.transpose(outg, (1, 0, 2)).reshape(HID, GC * NP)


def _cheb(t0t, bcounts, deg, wcat, cb):
    t0v = t0t.reshape(HID, NCHUNK, 1, GC * NP)
    bv = bcounts.reshape(NCHUNK, GC, NP * NP)
    degv = deg.reshape(NCHUNK, GC, NP)
    return pl.pallas_call(
        _k2_body,
        grid=(NCHUNK,),
        in_specs=[
            pl.BlockSpec((HID, 1, 1, GC * NP), lambda i: (0, i, 0, 0)),
            pl.BlockSpec((1, GC, NP * NP), lambda i: (i, 0, 0)),
            pl.BlockSpec((1, GC, NP), lambda i: (i, 0, 0)),
            pl.BlockSpec((3 * HID, HID), lambda i: (0, 0)),
            pl.BlockSpec((HID, 1), lambda i: (0, 0)),
        ],
        out_specs=pl.BlockSpec((HID, 1, 1, GC * NP), lambda i: (0, i, 0, 0)),
        out_shape=jax.ShapeDtypeStruct((HID, NCHUNK, 1, GC * NP), F32),
        compiler_params=pltpu.CompilerParams(dimension_semantics=("parallel",)),
    )(t0v, bv, degv, wcat, cb)


def _k3_body(x_ref, w_ref, b_ref, out_ref):
    x = x_ref[:, 0]
    x2 = jnp.concatenate([x[:, 0:36], x[:, 1:37], x[:, 2:38]], axis=0)
    x2 = x2.reshape(3 * HID, 36 * NP)
    y = jnp.dot(w_ref[...], x2, preferred_element_type=F32,
                precision=_PREC) + b_ref[...].reshape(192, 1)
    p = y[0:OUT]
    q = y[OUT:2 * OUT]
    r = y[2 * OUT:]
    t2 = jax.nn.relu(p * jax.nn.sigmoid(q) + r)
    out_ref[:, 0] = t2.reshape(OUT, 36, NP)


def _tconv2(t1t, w2all, b2):
    t1v = t1t.reshape(HID, B_, T_IN - 2, NP)
    return pl.pallas_call(
        _k3_body,
        grid=(B_,),
        in_specs=[
            pl.BlockSpec((HID, 1, T_IN - 2, NP), lambda b: (0, b, 0, 0)),
            pl.BlockSpec((192, 96), lambda b: (0, 0)),
            pl.BlockSpec((192, 1), lambda b: (0, 0)),
        ],
        out_specs=pl.BlockSpec((OUT, 1, 36, NP), lambda b: (0, b, 0, 0)),
        out_shape=jax.ShapeDtypeStruct((OUT, B_, 36, NP), F32),
        compiler_params=pltpu.CompilerParams(dimension_semantics=("parallel",)),
    )(t1v, w2all, b2)


def _k4a_body(x_ref, out_ref):
    b = pl.program_id(0)

    @pl.when(b == 0)
    def _():
        out_ref[...] = jnp.zeros_like(out_ref)

    x = x_ref[:, 0]
    s = jnp.sum(x, axis=(0, 1))
    ss = jnp.sum(x * x, axis=(0, 1))
    out_ref[0:1, :] += s[None, :]
    out_ref[1:2, :] += ss[None, :]


def _bn_stats(t2t):
    return pl.pallas_call(
        _k4a_body,
        grid=(B_,),
        in_specs=[pl.BlockSpec((OUT, 1, 36, NP), lambda b: (0, b, 0, 0))],
        out_specs=pl.BlockSpec((8, NP), lambda b: (0, 0)),
        out_shape=jax.ShapeDtypeStruct((8, NP), F32),
        compiler_params=pltpu.CompilerParams(dimension_semantics=("arbitrary",)),
    )(t2t)


def _k4b_body(x_ref, st_ref, bn_ref, out_ref):
    cnt = float(B_ * 36 * OUT)
    mean = st_ref[0:1, :] / cnt
    var = st_ref[1:2, :] / cnt - mean * mean
    gamma = bn_ref[0:1, :]
    beta = bn_ref[1:2, :]
    scale = gamma * lax.rsqrt(var + 1e-5)
    shift = beta - mean * scale
    x = x_ref[:, 0]
    xn = x * scale.reshape(1, 1, NP) + shift.reshape(1, 1, NP)
    s_all = jnp.sum(xn, axis=1)
    a0 = (s_all - xn[:, 34] - xn[:, 35]) * (1.0 / 34.0)
    a1 = (s_all - xn[:, 0] - xn[:, 35]) * (1.0 / 34.0)
    a2 = (s_all - xn[:, 0] - xn[:, 1]) * (1.0 / 34.0)
    a3 = jnp.stack([a0.T, a1.T, a2.T], axis=0)
    out_ref[...] = a3[:, 0:N_, :][None]


def _bn_pool(t2t, stats, bnp):
    return pl.pallas_call(
        _k4b_body,
        grid=(B_,),
        in_specs=[
            pl.BlockSpec((OUT, 1, 36, NP), lambda b: (0, b, 0, 0)),
            pl.BlockSpec((8, NP), lambda b: (0, 0)),
            pl.BlockSpec((8, NP), lambda b: (0, 0)),
        ],
        out_specs=pl.BlockSpec((1, 3, N_, OUT), lambda b: (b, 0, 0, 0)),
        out_shape=jax.ShapeDtypeStruct((B_, 3, N_, OUT), F32),
        compiler_params=pltpu.CompilerParams(dimension_semantics=("parallel",)),
    )(t2t, stats, bnp)


def _k4c_body(a_ref, wc_ref, cb_ref, fw_ref, fb_ref, out_ref):
    pooled = jnp.dot(a_ref[...], wc_ref[...], preferred_element_type=F32,
                     precision=_PREC) + cb_ref[...]
    out_ref[...] = jnp.dot(pooled, fw_ref[...], preferred_element_type=F32,
                           precision=_PREC) + fb_ref[...]


def _final(a3v, wc, c3b, f1_w, f1_b):
    return pl.pallas_call(
        _k4c_body,
        out_shape=jax.ShapeDtypeStruct((B_, N_ * OUT), F32),
    )(a3v, wc, c3b, f1_w, f1_b)


def kernel(X, edge_index, tc1_w1, tc1_b1, tc1_w2, tc1_b2, tc1_w3, tc1_b3,
           cheb_W, cheb_b, tc2_w1, tc2_b1, tc2_w2, tc2_b2, tc2_w3, tc2_b3,
           bn_gamma, bn_beta, conv3_w, conv3_b, f1_w, f1_b):
    ei_flat = edge_index.reshape(G_, 2, E_)
    bcounts, deg = _build_adj(ei_flat)

    xp = jnp.pad(X[..., 0], ((0, 0), (0, 0), (0, NP - N_)))
    w3 = jnp.concatenate([tc1_w1[:, 0, 0, :], tc1_w2[:, 0, 0, :],
                          tc1_w3[:, 0, 0, :]], axis=0)
    b3 = jnp.concatenate([tc1_b1, tc1_b2, tc1_b3])[:, None]
    t0t = _tconv1(xp, w3, b3)

    wcat = jnp.concatenate([(cheb_W[0] - cheb_W[2]).T, cheb_W[1].T,
                            2.0 * cheb_W[2].T], axis=0)
    t1t = _cheb(t0t.reshape(HID, G_ * NP), bcounts, deg, wcat,
                cheb_b[:, None])

    def _w2(w):
        return jnp.transpose(w[:, :, 0, :], (2, 1, 0)).reshape(3 * HID, OUT).T

    w2all = jnp.concatenate([_w2(tc2_w1), _w2(tc2_w2), _w2(tc2_w3)], axis=0)
    b2 = jnp.concatenate([tc2_b1, tc2_b2, tc2_b3])[:, None]
    t2t = _tconv2(t1t.reshape(HID, G_ * NP), w2all, b2)

    stats = _bn_stats(t2t)
    bnp = jnp.zeros((8, NP), F32)
    bnp = bnp.at[0, :N_].set(bn_gamma).at[1, :N_].set(bn_beta)
    a3v = _bn_pool(t2t, stats, bnp)

    wc = conv3_w.reshape(128, 3 * N_ * OUT).T
    return _final(a3v.reshape(B_, 3 * N_ * OUT), wc, conv3_b[None, :],
                  f1_w, f1_b[None, :])
